# baseline (speedup 1.0000x reference)
import math

import jax
import jax.numpy as jnp
from jax.experimental import pallas as pl
from jax.experimental.pallas import tpu as pltpu

_VOCAB = 128
_EMBED = 40
_NUM_HEADS = 4
_HEAD_DIM = _EMBED // _NUM_HEADS
_HIDDEN = 128
_NUM_LAYERS = 240
_SEQ_LEN = 8
_BATCH = 8
_LN_EPS = 1e-5
_M_ALL = _BATCH * _SEQ_LEN
_CORES = 2
_M = _M_ALL // _CORES
_HS = _NUM_HEADS * _M
_OUT_PAD = 128


def _layernorm(x, gamma, beta):
    mu = jnp.mean(x, axis=-1, keepdims=True)
    var = jnp.mean((x - mu) * (x - mu), axis=-1, keepdims=True)
    return (x - mu) * jax.lax.rsqrt(var + _LN_EPS) * gamma + beta


def _encoder_kernel(tok_ref, misc_ref, wqkv_ref, wo_ref, w1_ref, w2_ref,
                    vecs_ref, bout_ref, wout_ref, o_ref):
    emb = misc_ref[0:_VOCAB, :]
    pe = misc_ref[_VOCAB:_VOCAB + _SEQ_LEN, :]

    tok = tok_ref[...]
    vocab_iota = jax.lax.broadcasted_iota(jnp.int32, (_M, _VOCAB), 1)
    onehot = (vocab_iota == tok).astype(jnp.float32)
    x = jnp.dot(onehot, emb, preferred_element_type=jnp.float32)
    x = x + jnp.concatenate([pe] * (_M // _SEQ_LEN), axis=0)

    mrow = jax.lax.broadcasted_iota(jnp.int32, (_M, _HS), 0)
    mcol = jax.lax.broadcasted_iota(jnp.int32, (_M, _HS), 1)
    same_batch = (mrow // _SEQ_LEN) == ((mcol % _M) // _SEQ_LEN)
    hrow = jax.lax.broadcasted_iota(jnp.int32, (_HS, _EMBED), 0)
    hcol = jax.lax.broadcasted_iota(jnp.int32, (_HS, _EMBED), 1)
    head_mask = ((hrow // _M) == (hcol // _HEAD_DIM)).astype(jnp.float32)

    scale = 1.0 / math.sqrt(_HEAD_DIM)

    for l in range(_NUM_LAYERS):
        vec = vecs_ref[l]
        bqkv = vec[0:1, :3 * _EMBED]
        bo = vec[1:2, :_EMBED]
        b1 = vec[2:3, :_HIDDEN]
        b2 = vec[3:4, :_EMBED]
        g1 = vec[4:5, :_EMBED]
        be1 = vec[5:6, :_EMBED]
        g2 = vec[6:7, :_EMBED]
        be2 = vec[7:8, :_EMBED]

        qkv = jnp.dot(x, wqkv_ref[l], preferred_element_type=jnp.float32) + bqkv
        q = qkv[:, 0:_EMBED] * scale
        k = qkv[:, _EMBED:2 * _EMBED]
        v = qkv[:, 2 * _EMBED:3 * _EMBED]

        k_bd = jnp.concatenate([k] * _NUM_HEADS, axis=0) * head_mask
        v_bd = jnp.concatenate([v] * _NUM_HEADS, axis=0) * head_mask

        s = jax.lax.dot_general(q, k_bd, (((1,), (1,)), ((), ())),
                                preferred_element_type=jnp.float32)
        s = jnp.where(same_batch, s, -1e30)
        s = s - jnp.max(s, axis=-1, keepdims=True)
        p = jnp.exp(s)
        pt = p.astype(jnp.bfloat16).astype(jnp.float32)
        denom = jnp.concatenate(
            [jnp.broadcast_to(
                jnp.sum(pt[:, h * _M:(h + 1) * _M], axis=1, keepdims=True),
                (_M, _M)) for h in range(_NUM_HEADS)], axis=1)
        p = p / jnp.maximum(denom, 1e-20)
        attn = jnp.dot(p, v_bd, preferred_element_type=jnp.float32)
        attn = jnp.dot(attn, wo_ref[l], preferred_element_type=jnp.float32) + bo

        y = _layernorm(x + attn, g1, be1)
        h1 = jnp.maximum(jnp.dot(y, w1_ref[l], preferred_element_type=jnp.float32) + b1, 0.0)
        ff = jnp.dot(h1, w2_ref[l], preferred_element_type=jnp.float32) + b2
        x = _layernorm(y + ff, g2, be2)

    out = jnp.dot(x, wout_ref[...], preferred_element_type=jnp.float32) + bout_ref[...]
    o_ref[...] = out.astype(o_ref.dtype)


def kernel(tokens, misc, wqkv, wo, w1, w2, vecs, wout_pad):
    B, S = tokens.shape
    tok = tokens.reshape(B * S, 1).astype(jnp.int32)
    vecs3 = vecs[:_NUM_LAYERS * 8].reshape(_NUM_LAYERS, 8, 128)
    bout = vecs[_NUM_LAYERS * 8:_NUM_LAYERS * 8 + 1]

    def _full(arr):
        nd = arr.ndim
        return pl.BlockSpec(arr.shape, lambda i, _nd=nd: (0,) * _nd)

    out = pl.pallas_call(
        _encoder_kernel,
        out_shape=jax.ShapeDtypeStruct((_M_ALL, _OUT_PAD), jnp.float32),
        grid=(_CORES,),
        in_specs=[
            pl.BlockSpec((_M, 1), lambda i: (i, 0)),
            _full(misc), _full(wqkv), _full(wo), _full(w1), _full(w2),
            _full(vecs3), _full(bout), _full(wout_pad),
        ],
        out_specs=pl.BlockSpec((_M, _OUT_PAD), lambda i: (i, 0)),
        compiler_params=pltpu.CompilerParams(dimension_semantics=("parallel",)),
    )(tok, misc, wqkv, wo, w1, w2, vecs3, bout, wout_pad)
    return out[:, :_VOCAB].reshape(B, S, _VOCAB)


# speedup vs baseline: 1.4227x; 1.4227x over previous
import math

import jax
import jax.numpy as jnp
from jax.experimental import pallas as pl
from jax.experimental.pallas import tpu as pltpu

_VOCAB = 128
_EMBED = 40
_NUM_HEADS = 4
_HEAD_DIM = _EMBED // _NUM_HEADS
_HIDDEN = 128
_NUM_LAYERS = 240
_SEQ_LEN = 8
_BATCH = 8
_LN_EPS = 1e-5
_M_ALL = _BATCH * _SEQ_LEN
_CORES = 1
_M = _M_ALL // _CORES
_CHAINS = 2
_MC = _M // _CHAINS
_HSC = _NUM_HEADS * _MC
_OUT_PAD = 128


def _layernorm(x, gamma, beta):
    mu = jnp.mean(x, axis=-1, keepdims=True)
    var = jnp.mean((x - mu) * (x - mu), axis=-1, keepdims=True)
    return (x - mu) * jax.lax.rsqrt(var + _LN_EPS) * gamma + beta


def _encoder_kernel(tok_ref, misc_ref, wqkv_ref, wo_ref, w1_ref, w2_ref,
                    vecs_ref, bout_ref, wout_ref, o_ref):
    emb = misc_ref[0:_VOCAB, :]
    pe = misc_ref[_VOCAB:_VOCAB + _SEQ_LEN, :]

    mrow = jax.lax.broadcasted_iota(jnp.int32, (_MC, _HSC), 0)
    mcol = jax.lax.broadcasted_iota(jnp.int32, (_MC, _HSC), 1)
    same_batch = (mrow // _SEQ_LEN) == ((mcol % _MC) // _SEQ_LEN)
    hrow = jax.lax.broadcasted_iota(jnp.int32, (_HSC, _EMBED), 0)
    hcol = jax.lax.broadcasted_iota(jnp.int32, (_HSC, _EMBED), 1)
    head_mask = ((hrow // _MC) == (hcol // _HEAD_DIM)).astype(jnp.float32)

    scale = 1.0 / math.sqrt(_HEAD_DIM)

    tok = tok_ref[...]
    vocab_iota = jax.lax.broadcasted_iota(jnp.int32, (_M, _VOCAB), 1)
    onehot = (vocab_iota == tok).astype(jnp.float32)
    x0 = jnp.dot(onehot, emb, preferred_element_type=jnp.float32)
    x0 = x0 + jnp.concatenate([pe] * (_M // _SEQ_LEN), axis=0)
    xs = [x0[c * _MC:(c + 1) * _MC, :] for c in range(_CHAINS)]

    def one_layer(x, l):
        vec = vecs_ref[l]
        bqkv = vec[0:1, :3 * _EMBED]
        bo = vec[1:2, :_EMBED]
        b1 = vec[2:3, :_HIDDEN]
        b2 = vec[3:4, :_EMBED]
        g1 = vec[4:5, :_EMBED]
        be1 = vec[5:6, :_EMBED]
        g2 = vec[6:7, :_EMBED]
        be2 = vec[7:8, :_EMBED]

        qkv = jnp.dot(x, wqkv_ref[l], preferred_element_type=jnp.float32) + bqkv
        q = qkv[:, 0:_EMBED] * scale
        k = qkv[:, _EMBED:2 * _EMBED]
        v = qkv[:, 2 * _EMBED:3 * _EMBED]

        k_bd = jnp.concatenate([k] * _NUM_HEADS, axis=0) * head_mask
        v_bd = jnp.concatenate([v] * _NUM_HEADS, axis=0) * head_mask

        s = jax.lax.dot_general(q, k_bd, (((1,), (1,)), ((), ())),
                                preferred_element_type=jnp.float32)
        s = jnp.where(same_batch, s, -1e30)
        s = s - jnp.max(s, axis=-1, keepdims=True)
        p = jnp.exp(s)
        pt = p.astype(jnp.bfloat16).astype(jnp.float32)
        denom = jnp.concatenate(
            [jnp.broadcast_to(
                jnp.maximum(jnp.sum(pt[:, h * _MC:(h + 1) * _MC], axis=1,
                                    keepdims=True), 1e-20),
                (_MC, _MC)) for h in range(_NUM_HEADS)], axis=1)
        p = p / denom
        attn = jnp.dot(p, v_bd, preferred_element_type=jnp.float32)
        attn = jnp.dot(attn, wo_ref[l], preferred_element_type=jnp.float32) + bo

        y = _layernorm(x + attn, g1, be1)
        h1 = jnp.maximum(jnp.dot(y, w1_ref[l], preferred_element_type=jnp.float32) + b1, 0.0)
        ff = jnp.dot(h1, w2_ref[l], preferred_element_type=jnp.float32) + b2
        return _layernorm(y + ff, g2, be2)

    for l in range(_NUM_LAYERS):
        xs = [one_layer(x, l) for x in xs]

    wout = wout_ref[...]
    bout = bout_ref[...]
    for c in range(_CHAINS):
        out = jnp.dot(xs[c], wout, preferred_element_type=jnp.float32) + bout
        o_ref[c * _MC:(c + 1) * _MC, :] = out.astype(o_ref.dtype)


def kernel(tokens, misc, wqkv, wo, w1, w2, vecs, wout_pad):
    B, S = tokens.shape
    tok = tokens.reshape(B * S, 1).astype(jnp.int32)
    vecs3 = vecs[:_NUM_LAYERS * 8].reshape(_NUM_LAYERS, 8, 128)
    bout = vecs[_NUM_LAYERS * 8:_NUM_LAYERS * 8 + 1]

    def _full(arr):
        nd = arr.ndim
        return pl.BlockSpec(arr.shape, lambda i, _nd=nd: (0,) * _nd)

    out = pl.pallas_call(
        _encoder_kernel,
        out_shape=jax.ShapeDtypeStruct((_M_ALL, _OUT_PAD), jnp.float32),
        grid=(_CORES,),
        in_specs=[
            pl.BlockSpec((_M, 1), lambda i: (i, 0)),
            _full(misc), _full(wqkv), _full(wo), _full(w1), _full(w2),
            _full(vecs3), _full(bout), _full(wout_pad),
        ],
        out_specs=pl.BlockSpec((_M, _OUT_PAD), lambda i: (i, 0)),
        compiler_params=pltpu.CompilerParams(dimension_semantics=("arbitrary",)),
    )(tok, misc, wqkv, wo, w1, w2, vecs3, bout, wout_pad)
    return out[:, :_VOCAB].reshape(B, S, _VOCAB)


# speedup vs baseline: 1.9547x; 1.3739x over previous
import math

import jax
import jax.numpy as jnp
from jax.experimental import pallas as pl
from jax.experimental.pallas import tpu as pltpu

_VOCAB = 128
_EMBED = 40
_NUM_HEADS = 4
_HEAD_DIM = _EMBED // _NUM_HEADS
_HIDDEN = 128
_NUM_LAYERS = 240
_SEQ_LEN = 8
_BATCH = 8
_LN_EPS = 1e-5
_M_ALL = _BATCH * _SEQ_LEN
_CORES = 1
_M = _M_ALL // _CORES
_CHAINS = 2
_MC = _M // _CHAINS
_HSC = _NUM_HEADS * _MC
_OUT_PAD = 128


def _layernorm(x, gamma, beta):
    mu = jnp.mean(x, axis=-1, keepdims=True)
    var = jnp.mean((x - mu) * (x - mu), axis=-1, keepdims=True)
    return (x - mu) * jax.lax.rsqrt(var + _LN_EPS) * gamma + beta


def _encoder_kernel(tok_ref, misc_ref, wqkv_ref, wo_ref, w1_ref, w2_ref,
                    vecs_ref, bout_ref, wout_ref, o_ref):
    emb = misc_ref[0:_VOCAB, :]
    pe = misc_ref[_VOCAB:_VOCAB + _SEQ_LEN, :]

    mrow = jax.lax.broadcasted_iota(jnp.int32, (_MC, _HSC), 0)
    mcol = jax.lax.broadcasted_iota(jnp.int32, (_MC, _HSC), 1)
    same_batch = (mrow // _SEQ_LEN) == ((mcol % _MC) // _SEQ_LEN)
    hrow = jax.lax.broadcasted_iota(jnp.int32, (_HSC, _EMBED), 0)
    hcol = jax.lax.broadcasted_iota(jnp.int32, (_HSC, _EMBED), 1)
    head_mask = ((hrow // _MC) == (hcol // _HEAD_DIM)).astype(jnp.float32)
    brow = jax.lax.broadcasted_iota(jnp.int32, (_HSC, _HSC), 0)
    bcol = jax.lax.broadcasted_iota(jnp.int32, (_HSC, _HSC), 1)
    block_ones = ((brow // _SEQ_LEN) == (bcol // _SEQ_LEN)).astype(jnp.float32)

    scale = 1.0 / math.sqrt(_HEAD_DIM)

    tok = tok_ref[...]
    vocab_iota = jax.lax.broadcasted_iota(jnp.int32, (_M, _VOCAB), 1)
    onehot = (vocab_iota == tok).astype(jnp.float32)
    x0 = jnp.dot(onehot, emb, preferred_element_type=jnp.float32)
    x0 = x0 + jnp.concatenate([pe] * (_M // _SEQ_LEN), axis=0)
    xs = [x0[c * _MC:(c + 1) * _MC, :] for c in range(_CHAINS)]

    zpad = vecs_ref[0][1:2, 60:61]

    def one_layer(x, l):
        vec = vecs_ref[l]
        bqkv = vec[0:1, :3 * _EMBED]
        bo = vec[1:2, :_EMBED]
        b1 = vec[2:3, :_HIDDEN]
        b2 = vec[3:4, :_EMBED]
        g1 = vec[4:5, :_EMBED]
        be1 = vec[5:6, :_EMBED]
        g2 = vec[6:7, :_EMBED]
        be2 = vec[7:8, :_EMBED]

        qkv = jnp.dot(x, wqkv_ref[l], preferred_element_type=jnp.float32) + bqkv
        q = qkv[:, 0:_EMBED] * scale
        k = qkv[:, _EMBED:2 * _EMBED]
        v = qkv[:, 2 * _EMBED:3 * _EMBED]

        k_bd = jnp.concatenate([k] * _NUM_HEADS, axis=0) * head_mask
        v_bd = jnp.concatenate([v] * _NUM_HEADS, axis=0) * head_mask

        s = jax.lax.dot_general(q, k_bd, (((1,), (1,)), ((), ())),
                                preferred_element_type=jnp.float32)
        s = jnp.where(same_batch, s, -1e30)
        s = s - jnp.max(s, axis=-1, keepdims=True)
        p = jnp.exp(s)
        denom = jnp.dot(p, block_ones, preferred_element_type=jnp.float32)
        p = p / jnp.maximum(denom, 1e-20)
        attn = jnp.dot(p, v_bd, preferred_element_type=jnp.float32)
        attn = jnp.dot(attn, wo_ref[l], preferred_element_type=jnp.float32) + bo

        y = _layernorm(x + attn, g1, be1)
        h1 = jnp.maximum(jnp.dot(y, w1_ref[l], preferred_element_type=jnp.float32) + b1, 0.0)
        ff = jnp.dot(h1, w2_ref[l], preferred_element_type=jnp.float32) + b2
        return _layernorm(y + ff, g2, be2), p[0:1, 0:1]

    for l in range(_NUM_LAYERS):
        new_xs = []
        probe = None
        for c in range(_CHAINS):
            xin = xs[c] if probe is None else xs[c] + zpad * probe
            xnew, probe = one_layer(xin, l)
            new_xs.append(xnew)
        xs = new_xs

    wout = wout_ref[...]
    bout = bout_ref[...]
    for c in range(_CHAINS):
        out = jnp.dot(xs[c], wout, preferred_element_type=jnp.float32) + bout
        o_ref[c * _MC:(c + 1) * _MC, :] = out.astype(o_ref.dtype)


def kernel(tokens, misc, wqkv, wo, w1, w2, vecs, wout_pad):
    B, S = tokens.shape
    tok = tokens.reshape(B * S, 1).astype(jnp.int32)
    vecs3 = vecs[:_NUM_LAYERS * 8].reshape(_NUM_LAYERS, 8, 128)
    bout = vecs[_NUM_LAYERS * 8:_NUM_LAYERS * 8 + 1]

    def _full(arr):
        nd = arr.ndim
        return pl.BlockSpec(arr.shape, lambda i, _nd=nd: (0,) * _nd)

    out = pl.pallas_call(
        _encoder_kernel,
        out_shape=jax.ShapeDtypeStruct((_M_ALL, _OUT_PAD), jnp.float32),
        grid=(_CORES,),
        in_specs=[
            pl.BlockSpec((_M, 1), lambda i: (i, 0)),
            _full(misc), _full(wqkv), _full(wo), _full(w1), _full(w2),
            _full(vecs3), _full(bout), _full(wout_pad),
        ],
        out_specs=pl.BlockSpec((_M, _OUT_PAD), lambda i: (i, 0)),
        compiler_params=pltpu.CompilerParams(dimension_semantics=("arbitrary",)),
    )(tok, misc, wqkv, wo, w1, w2, vecs3, bout, wout_pad)
    return out[:, :_VOCAB].reshape(B, S, _VOCAB)
